# Initial kernel scaffold; baseline (speedup 1.0000x reference)
import jax
import jax.numpy as jnp
from jax.experimental import pallas as pl


def kernel(x, edge_index, W1, b1, W2, b2):
    raise NotImplementedError("write your pallas kernel here")


# speedup vs baseline: 4.9486x; 4.9486x over previous
import functools

import jax
import jax.numpy as jnp
from jax import lax
from jax.experimental import pallas as pl
from jax.experimental.pallas import tpu as pltpu
from jax.experimental.pallas import tpu_sc as plsc

N = 10000
D = 128
E = 320000
K = 10
ALPHA = 0.1
SCALE = 1.8

NC = 2
NS = 16
NW = NC * NS
NP = 10240
ROWS_PER_TILE = NP // NS
CH = 128
ET = E + N
EPW = 10496
EPAD = NW * EPW
NCHUNK = EPW // CH

_mesh = plsc.VectorSubcoreMesh(core_axis_name="c", subcore_axis_name="s")


@functools.partial(
    pl.kernel,
    mesh=_mesh,
    out_type=jax.ShapeDtypeStruct((2 * NP, D), jnp.float32),
    scratch_types=[
        pltpu.VMEM_SHARED((NP, D), jnp.float32),
        pltpu.VMEM((CH, D), jnp.float32),
        pltpu.VMEM((CH, D), jnp.float32),
        pltpu.VMEM((CH,), jnp.int32),
        pltpu.VMEM((CH,), jnp.int32),
        pltpu.VMEM((CH,), jnp.int32),
        pltpu.VMEM((CH,), jnp.int32),
        pltpu.SemaphoreType.DMA,
    ],
)
def _sc_edge_pass(g_hbm, sidx_hbm, didx_hbm, zslab_hbm, out_hbm,
                  acc, rows_a, rows_b, sidx_a, didx_a, sidx_b, didx_b, sem):
    c = lax.axis_index("c")
    s_ = lax.axis_index("s")
    wid = s_ * NC + c

    pltpu.sync_copy(zslab_hbm, acc.at[pl.ds(s_ * ROWS_PER_TILE, ROWS_PER_TILE)])
    plsc.subcore_barrier()

    def _load_idx(chunk, si, di):
        base = wid * EPW + chunk * CH
        pltpu.sync_copy(sidx_hbm.at[pl.ds(base, CH)], si)
        pltpu.sync_copy(didx_hbm.at[pl.ds(base, CH)], di)

    _load_idx(0, sidx_a, didx_a)
    pltpu.async_copy(g_hbm.at[sidx_a], rows_a, sem)

    def body(j2, carry):
        c0 = 2 * j2
        _load_idx(c0 + 1, sidx_b, didx_b)
        pltpu.make_async_copy(g_hbm.at[sidx_a], rows_a, sem).wait()
        pltpu.async_copy(g_hbm.at[sidx_b], rows_b, sem)
        pltpu.sync_copy(rows_a, acc.at[didx_a], add=True)
        c2 = jnp.minimum(c0 + 2, NCHUNK - 1)
        _load_idx(c2, sidx_a, didx_a)
        pltpu.make_async_copy(g_hbm.at[sidx_b], rows_b, sem).wait()
        pltpu.async_copy(g_hbm.at[sidx_a], rows_a, sem)
        pltpu.sync_copy(rows_b, acc.at[didx_b], add=True)
        return carry

    lax.fori_loop(0, NCHUNK // 2, body, 0)
    pltpu.make_async_copy(g_hbm.at[sidx_a], rows_a, sem).wait()
    plsc.subcore_barrier()

    pltpu.sync_copy(
        acc.at[pl.ds(s_ * ROWS_PER_TILE, ROWS_PER_TILE)],
        out_hbm.at[pl.ds(c * NP + s_ * ROWS_PER_TILE, ROWS_PER_TILE)],
    )


BR = 2000
GRID = N // BR


def _tc_init_body(x_ref, w1_ref, b1_ref, w2_ref, b2_ref, p0_ref, p1_ref,
                  xlin_ref, h0_ref, dinv_ref, gx_ref, gh_ref):
    x = x_ref[...]
    xlin = lax.dot_general(
        x, w1_ref[...], (((1,), (1,)), ((), ())),
        precision=lax.Precision.HIGHEST) + b1_ref[...]
    h = lax.dot_general(
        x, w2_ref[...], (((1,), (1,)), ((), ())),
        precision=lax.Precision.HIGHEST) + b2_ref[...]
    nrm = jnp.maximum(jnp.sqrt(jnp.sum(h * h, axis=1, keepdims=True)), 1e-12)
    h0 = h / nrm * SCALE
    deg = p0_ref[:, 0:1] + p1_ref[:, 0:1]
    dinv = jnp.where(deg > 0, lax.rsqrt(jnp.maximum(deg, 1e-12)), 0.0)
    dinvb = jnp.broadcast_to(dinv, (BR, D))
    xlin_ref[...] = xlin
    h0_ref[...] = h0
    dinv_ref[...] = dinvb
    gx_ref[...] = dinvb * xlin
    gh_ref[...] = dinvb * h0


def _tc_combine_body(p0_ref, p1_ref, dinv_ref, x0_ref, h_ref, g_ref):
    agg = p0_ref[...] + p1_ref[...]
    h = (1.0 - ALPHA) * (dinv_ref[...] * agg) + ALPHA * x0_ref[...]
    h_ref[...] = h
    g_ref[...] = dinv_ref[...] * h


_f32 = jnp.float32
_row_spec = pl.BlockSpec((BR, D), lambda i: (i, 0))
_w_spec = pl.BlockSpec((D, D), lambda i: (0, 0))
_b_spec = pl.BlockSpec((1, D), lambda i: (0, 0))
_row_out = jax.ShapeDtypeStruct((N, D), _f32)


def _tc_init(x, W1, b1, W2, b2, p):
    return pl.pallas_call(
        _tc_init_body,
        grid=(GRID,),
        in_specs=[_row_spec, _w_spec, _b_spec, _w_spec, _b_spec,
                  _row_spec, _row_spec],
        out_specs=[_row_spec] * 5,
        out_shape=[_row_out] * 5,
    )(x, W1, b1.reshape(1, D), W2, b2.reshape(1, D),
      p[:N], p[NP:NP + N])


def _tc_combine(p, dinvb, x0):
    return pl.pallas_call(
        _tc_combine_body,
        grid=(GRID,),
        in_specs=[_row_spec] * 4,
        out_specs=[_row_spec] * 2,
        out_shape=[_row_out] * 2,
    )(p[:N], p[NP:NP + N], dinvb, x0)


def kernel(x, edge_index, W1, b1, W2, b2):
    src = edge_index[0].astype(jnp.int32)
    dst = edge_index[1].astype(jnp.int32)
    loop = jnp.arange(N, dtype=jnp.int32)
    pad = EPAD - ET
    s_all = jnp.concatenate([src, loop, jnp.zeros((pad,), jnp.int32)])
    d_all = jnp.concatenate([dst, loop, jnp.full((pad,), NP - 1, jnp.int32)])
    zslab = jnp.zeros((ROWS_PER_TILE, D), _f32)

    ones_g = jnp.ones((N, D), _f32)
    p = _sc_edge_pass(ones_g, s_all, d_all, zslab)
    xlin, h0, dinvb, g, gh = _tc_init(x, W1, b1, W2, b2, p)

    x_ = xlin
    for _ in range(K):
        p = _sc_edge_pass(g, s_all, d_all, zslab)
        x_, g = _tc_combine(p, dinvb, xlin)

    h = h0
    g = gh
    for _ in range(K):
        p = _sc_edge_pass(g, s_all, d_all, zslab)
        h, g = _tc_combine(p, dinvb, h0)

    return (h, x_)
